# Initial kernel scaffold; baseline (speedup 1.0000x reference)
import jax
import jax.numpy as jnp
from jax.experimental import pallas as pl


def kernel(x, edge_index, edge_weight, W, b):
    raise NotImplementedError("write your pallas kernel here")


# speedup vs baseline: 3.4489x; 3.4489x over previous
import functools

import jax
import jax.numpy as jnp
from jax import lax
from jax.experimental import pallas as pl
from jax.experimental.pallas import tpu as pltpu
from jax.experimental.pallas import tpu_sc as plsc

NC = 2
NS = 16
L = 16
NW = NC * NS
EB = 128


def _mm_body(x_ref, w_ref, o_ref):
    o_ref[...] = jnp.dot(x_ref[...], w_ref[...],
                         preferred_element_type=jnp.float32)


def _matmul(x, W):
    n, d_in = x.shape
    d_out = W.shape[1]
    bm = 1000 if n % 1000 == 0 else 8
    n_pad = ((n + bm - 1) // bm) * bm
    if n_pad != n:
        x = jnp.pad(x, ((0, n_pad - n), (0, 0)))
    h = pl.pallas_call(
        _mm_body,
        grid=(n_pad // bm,),
        in_specs=[
            pl.BlockSpec((bm, d_in), lambda i: (i, 0)),
            pl.BlockSpec((d_in, d_out), lambda i: (0, 0)),
        ],
        out_specs=pl.BlockSpec((bm, d_out), lambda i: (i, 0)),
        out_shape=jax.ShapeDtypeStruct((n_pad, d_out), jnp.float32),
    )(x, W)
    return h[:n] if n_pad != n else h


def _comb_body(p0_ref, p1_ref, b_ref, o_ref):
    o_ref[...] = p0_ref[...] + p1_ref[...] + b_ref[...]


def _combine(p0, p1, b):
    n, d = p0.shape
    bm = 1000 if n % 1000 == 0 else 8
    n_pad = ((n + bm - 1) // bm) * bm
    if n_pad != n:
        p0 = jnp.pad(p0, ((0, n_pad - n), (0, 0)))
        p1 = jnp.pad(p1, ((0, n_pad - n), (0, 0)))
    out = pl.pallas_call(
        _comb_body,
        grid=(n_pad // bm,),
        in_specs=[
            pl.BlockSpec((bm, d), lambda i: (i, 0)),
            pl.BlockSpec((bm, d), lambda i: (i, 0)),
            pl.BlockSpec((1, d), lambda i: (0, 0)),
        ],
        out_specs=pl.BlockSpec((bm, d), lambda i: (i, 0)),
        out_shape=jax.ShapeDtypeStruct((n_pad, d), jnp.float32),
    )(p0, p1, b.reshape(1, d))
    return out[:n] if n_pad != n else out


def _make_edge_kernel(n_acc, d, e_pad):
    ew = e_pad // NW
    nb = ew // EB
    rows_per_tile = n_acc // NS
    mesh = plsc.VectorSubcoreMesh(core_axis_name="c", subcore_axis_name="s")

    @functools.partial(
        pl.kernel,
        mesh=mesh,
        out_type=jax.ShapeDtypeStruct((NC, n_acc, d), jnp.float32),
        scratch_types=[
            pltpu.VMEM_SHARED((n_acc, d), jnp.float32),
            pltpu.VMEM((EB,), jnp.int32),
            pltpu.VMEM((EB,), jnp.int32),
            pltpu.VMEM((EB,), jnp.float32),
            pltpu.VMEM((EB, d), jnp.float32),
            pltpu.SemaphoreType.DMA,
        ],
    )
    def edge_kernel(h_hbm, src_hbm, dst_hbm, w_hbm, part_hbm,
                    acc, src_v, dst_v, w_v, rows_v, gsem):
        cid = lax.axis_index("c")
        sid = lax.axis_index("s")
        wid = sid * NC + cid

        @pl.loop(0, EB)
        def _zero_rows(r):
            for j in range(d // L):
                rows_v[r, pl.ds(j * L, L)] = jnp.zeros((L,), jnp.float32)

        stripe0 = sid * rows_per_tile
        done = 0
        while done < rows_per_tile:
            step = min(EB, rows_per_tile - done)
            pltpu.sync_copy(rows_v.at[pl.ds(0, step)],
                            acc.at[pl.ds(stripe0 + done, step)])
            done += step
        plsc.subcore_barrier()

        base = wid * ew

        @pl.loop(0, nb)
        def _batch(i):
            off = base + i * EB
            pltpu.sync_copy(src_hbm.at[pl.ds(off, EB)], src_v)
            pltpu.sync_copy(dst_hbm.at[pl.ds(off, EB)], dst_v)
            pltpu.sync_copy(w_hbm.at[pl.ds(off, EB)], w_v)
            pltpu.async_copy(h_hbm.at[src_v], rows_v, gsem).wait()

            @pl.loop(0, EB // L)
            def _scale(g):
                wchunk = w_v[pl.ds(g * L, L)]
                for k in range(L):
                    wv = jnp.full((L,), wchunk[k], jnp.float32)
                    e = g * L + k
                    for j in range(d // L):
                        rows_v[e, pl.ds(j * L, L)] = (
                            rows_v[e, pl.ds(j * L, L)] * wv)

            pltpu.sync_copy(rows_v, acc.at[dst_v], add=True)

        plsc.subcore_barrier()
        done = 0
        while done < rows_per_tile:
            step = min(EB, rows_per_tile - done)
            pltpu.sync_copy(acc.at[pl.ds(stripe0 + done, step)],
                            part_hbm.at[cid, pl.ds(stripe0 + done, step)])
            done += step

    return edge_kernel


def kernel(x, edge_index, edge_weight, W, b):
    n, d_in = x.shape
    d = W.shape[1]
    e = edge_index.shape[1]

    h = _matmul(x, W)

    e_pad = ((e + NW * EB - 1) // (NW * EB)) * (NW * EB)
    src = edge_index[1]
    dst = edge_index[0]
    w = edge_weight
    if e_pad != e:
        pad = e_pad - e
        src = jnp.pad(src, (0, pad))
        dst = jnp.pad(dst, (0, pad))
        w = jnp.pad(w, (0, pad))

    n_acc = ((n + NS * 8 - 1) // (NS * 8)) * (NS * 8)

    part = _make_edge_kernel(n_acc, d, e_pad)(h, src, dst, w)
    p0 = part[0, :n]
    p1 = part[1, :n]
    return _combine(p0, p1, b)
